# Initial kernel scaffold; baseline (speedup 1.0000x reference)
import jax
import jax.numpy as jnp
from jax.experimental import pallas as pl


def kernel(x, edge_index, W1, b1, W2, b2):
    raise NotImplementedError("write your pallas kernel here")


# speedup vs baseline: 2.3560x; 2.3560x over previous
import dataclasses
import functools

import jax
import jax.numpy as jnp
from jax import lax
from jax.experimental import pallas as pl
from jax.experimental.pallas import tpu as pltpu
from jax.experimental.pallas import tpu_sc as plsc

N = 10000
D = 128
E = 320000

NC = 2
NS = 16
NW = NC * NS

NPAD = 10240
NB = NPAD // 128
CH = 128
EPAD = NW * 80 * CH
RPT = EPAD // (NW * CH)
ZB = NPAD // (NS * CH)


def _sc_mesh():
    return plsc.VectorSubcoreMesh(
        core_axis_name="c", subcore_axis_name="s", num_cores=NC, num_subcores=NS
    )


def _sc_compiler_params():
    cp = pltpu.CompilerParams()
    if "needs_layout_passes" in pltpu.CompilerParams.__dataclass_fields__:
        cp = dataclasses.replace(cp, needs_layout_passes=False)
    return cp


@jax.jit
def _sc_degrees(src1d, dst1d):
    @functools.partial(
        pl.kernel,
        out_type=(
            jax.ShapeDtypeStruct((NW, NB, 128), jnp.float32),
            jax.ShapeDtypeStruct((NW, NB, 128), jnp.float32),
        ),
        mesh=_sc_mesh(),
        compiler_params=_sc_compiler_params(),
        scratch_types=[
            pltpu.VMEM((CH,), jnp.int32),
            pltpu.VMEM((CH,), jnp.int32),
            pltpu.VMEM((NB, 128), jnp.float32),
            pltpu.VMEM((NB, 128), jnp.float32),
        ],
    )
    def deg_kernel(src_hbm, dst_hbm, odeg_hbm, ideg_hbm,
                   svec, dvec, hist_o, hist_i):
        cid = lax.axis_index("c")
        sid = lax.axis_index("s")
        tile = cid * NS + sid

        @pl.loop(0, NB)
        def _(i):
            @pl.loop(0, 128, step=16)
            def _(j):
                z = jnp.zeros((16,), jnp.float32)
                hist_o[i, pl.ds(j, 16)] = z
                hist_i[i, pl.ds(j, 16)] = z

        ones16 = jnp.ones((16,), jnp.float32)

        @pl.loop(0, RPT)
        def _(b):
            base = (tile * RPT + b) * CH
            pltpu.sync_copy(src_hbm.at[pl.ds(base, CH)], svec)
            pltpu.sync_copy(dst_hbm.at[pl.ds(base, CH)], dvec)
            for k in range(CH // 16):
                s16 = svec[pl.ds(k * 16, 16)]
                d16 = dvec[pl.ds(k * 16, 16)]
                plsc.addupdate_scatter(hist_o, [s16 >> 7, s16 & 127], ones16)
                plsc.addupdate_scatter(hist_i, [d16 >> 7, d16 & 127], ones16)

        pltpu.sync_copy(hist_o, odeg_hbm.at[tile])
        pltpu.sync_copy(hist_i, ideg_hbm.at[tile])

    return deg_kernel(src1d, dst1d)


@jax.jit
def _sc_aggregate(h, src1d, dst1d, rows1d, zeros_h):
    @functools.partial(
        pl.kernel,
        out_type=jax.ShapeDtypeStruct((NC, NPAD, D), jnp.float32),
        mesh=_sc_mesh(),
        scratch_types=[
            pltpu.VMEM((CH,), jnp.int32),
            pltpu.VMEM((CH,), jnp.int32),
            pltpu.VMEM((CH,), jnp.int32),
            pltpu.VMEM((CH, D), jnp.float32),
            pltpu.VMEM_SHARED((NPAD, D), jnp.float32),
        ],
    )
    def agg_kernel(h_hbm, src_hbm, dst_hbm, rows_hbm, zeros_hbm, out_hbm,
                   svec, dvec, rowvec, rows_v, acc):
        cid = lax.axis_index("c")
        sid = lax.axis_index("s")
        tile = cid * NS + sid

        pltpu.sync_copy(zeros_hbm, rows_v)

        @pl.loop(0, ZB)
        def _(j):
            pltpu.sync_copy(rows_hbm.at[pl.ds((sid * ZB + j) * CH, CH)], rowvec)
            pltpu.sync_copy(rows_v, acc.at[rowvec])

        plsc.subcore_barrier()

        @pl.loop(0, RPT)
        def _(b):
            base = (tile * RPT + b) * CH
            pltpu.sync_copy(src_hbm.at[pl.ds(base, CH)], svec)
            pltpu.sync_copy(dst_hbm.at[pl.ds(base, CH)], dvec)
            pltpu.sync_copy(h_hbm.at[svec], rows_v)
            pltpu.sync_copy(rows_v, acc.at[dvec], add=True)

        plsc.subcore_barrier()

        @pl.loop(0, ZB)
        def _(j):
            pltpu.sync_copy(rows_hbm.at[pl.ds((sid * ZB + j) * CH, CH)], rowvec)
            pltpu.sync_copy(acc.at[rowvec], rows_v)
            pltpu.sync_copy(rows_v, out_hbm.at[cid, pl.ds((sid * ZB + j) * CH, CH)])

    return agg_kernel(h, src1d, dst1d, rows1d, zeros_h)


@jax.jit
def _tc_scales(odeg, ideg):
    def body(od_ref, id_ref, so_ref, si_ref):
        so = lax.rsqrt(jnp.maximum(jnp.sum(od_ref[...], axis=0), 1.0))
        si = lax.rsqrt(jnp.maximum(jnp.sum(id_ref[...], axis=0), 1.0))
        so_ref[...] = so.reshape(NB, 1, 128)
        si_ref[...] = si.reshape(NB, 1, 128)

    return pl.pallas_call(
        body,
        out_shape=(
            jax.ShapeDtypeStruct((NB, 1, 128), jnp.float32),
            jax.ShapeDtypeStruct((NB, 1, 128), jnp.float32),
        ),
    )(odeg, ideg)


@jax.jit
def _tc_in_matmul(x_pad, so, eye, W1):
    def body(x_ref, s_ref, e_ref, w_ref, o_ref):
        diag = e_ref[...] * s_ref[0]
        xs = jnp.dot(diag, x_ref[...], preferred_element_type=jnp.float32)
        o_ref[...] = jnp.dot(xs, w_ref[...], preferred_element_type=jnp.float32)

    return pl.pallas_call(
        body,
        grid=(NB,),
        in_specs=[
            pl.BlockSpec((128, D), lambda i: (i, 0)),
            pl.BlockSpec((1, 1, 128), lambda i: (i, 0, 0)),
            pl.BlockSpec((D, D), lambda i: (0, 0)),
            pl.BlockSpec((D, D), lambda i: (0, 0)),
        ],
        out_specs=pl.BlockSpec((128, D), lambda i: (i, 0)),
        out_shape=jax.ShapeDtypeStruct((NPAD, D), jnp.float32),
    )(x_pad, so, eye, W1)


@jax.jit
def _tc_mid(agg1, si, so, eye, b1, W2):
    def body(a_ref, si_ref, so_ref, e_ref, b_ref, w_ref, o_ref):
        a = a_ref[0] + a_ref[1]
        diag_i = e_ref[...] * si_ref[0]
        y = jnp.maximum(
            jnp.dot(diag_i, a, preferred_element_type=jnp.float32) + b_ref[...],
            0.0,
        )
        diag_o = e_ref[...] * so_ref[0]
        ys = jnp.dot(diag_o, y, preferred_element_type=jnp.float32)
        o_ref[...] = jnp.dot(ys, w_ref[...], preferred_element_type=jnp.float32)

    return pl.pallas_call(
        body,
        grid=(NB,),
        in_specs=[
            pl.BlockSpec((NC, 128, D), lambda i: (0, i, 0)),
            pl.BlockSpec((1, 1, 128), lambda i: (i, 0, 0)),
            pl.BlockSpec((1, 1, 128), lambda i: (i, 0, 0)),
            pl.BlockSpec((D, D), lambda i: (0, 0)),
            pl.BlockSpec((1, D), lambda i: (0, 0)),
            pl.BlockSpec((D, D), lambda i: (0, 0)),
        ],
        out_specs=pl.BlockSpec((128, D), lambda i: (i, 0)),
        out_shape=jax.ShapeDtypeStruct((NPAD, D), jnp.float32),
    )(agg1, si, so, eye, b1, W2)


@jax.jit
def _tc_final(agg2, si, eye, b2):
    def body(a_ref, si_ref, e_ref, b_ref, o_ref):
        a = a_ref[0] + a_ref[1]
        diag_i = e_ref[...] * si_ref[0]
        o_ref[...] = (
            jnp.dot(diag_i, a, preferred_element_type=jnp.float32) + b_ref[...]
        )

    return pl.pallas_call(
        body,
        grid=(NB,),
        in_specs=[
            pl.BlockSpec((NC, 128, D), lambda i: (0, i, 0)),
            pl.BlockSpec((1, 1, 128), lambda i: (i, 0, 0)),
            pl.BlockSpec((D, D), lambda i: (0, 0)),
            pl.BlockSpec((1, D), lambda i: (0, 0)),
        ],
        out_specs=pl.BlockSpec((128, D), lambda i: (i, 0)),
        out_shape=jax.ShapeDtypeStruct((NPAD, D), jnp.float32),
    )(agg2, si, eye, b2)


def kernel(x, edge_index, W1, b1, W2, b2):
    x = x.astype(jnp.float32)
    src = edge_index[0].astype(jnp.int32)
    dst = edge_index[1].astype(jnp.int32)
    pad = jnp.full((EPAD - E,), NPAD - 1, jnp.int32)
    src1d = jnp.concatenate([src, pad])
    dst1d = jnp.concatenate([dst, pad])
    rows1d = jnp.arange(NPAD, dtype=jnp.int32)
    zeros_h = jnp.zeros((CH, D), jnp.float32)
    eye = jnp.eye(D, dtype=jnp.float32)
    x_pad = jnp.pad(x, ((0, NPAD - N), (0, 0)))

    odeg, ideg = _sc_degrees(src1d, dst1d)
    so, si = _tc_scales(odeg, ideg)
    h1 = _tc_in_matmul(x_pad, so, eye, W1)
    agg1 = _sc_aggregate(h1, src1d, dst1d, rows1d, zeros_h)
    h2 = _tc_mid(agg1, si, so, eye, b1.reshape(1, D), W2)
    agg2 = _sc_aggregate(h2, src1d, dst1d, rows1d, zeros_h)
    out = _tc_final(agg2, si, eye, b2.reshape(1, D))
    return out[:N]
